# baseline (speedup 1.0000x reference)
import functools

import jax
import jax.numpy as jnp
from jax import lax
from jax.experimental import pallas as pl
from jax.experimental.pallas import tpu as pltpu
from jax.experimental.pallas import tpu_sc as plsc

BATCH = 16384
MF_DIM = 32
MLP_HALF = 16
NC = 2
NS = 16
NW = NC * NS
B_PER_W = BATCH // NW
CHUNK = 128
N_CHUNK = B_PER_W // CHUNK


def _sc_gather_body(user_r, item_r, mfu_t, mfi_t, mlpu_t, mlpi_t,
                    out_mfu, out_mfi, out_mlpu, out_mlpi,
                    idx_u, idx_i, buf_mfu, buf_mfi, buf_mlpu, buf_mlpi, sem):
    wid = lax.axis_index("s") * NC + lax.axis_index("c")
    base = wid * B_PER_W
    pltpu.sync_copy(user_r.at[wid], idx_u)
    pltpu.sync_copy(item_r.at[wid], idx_i)
    copies = []
    for j in range(N_CHUNK):
        dst = pl.ds(j * CHUNK, CHUNK)
        copies.append(pltpu.async_copy(mfu_t.at[idx_u.at[j]], buf_mfu.at[dst], sem))
        copies.append(pltpu.async_copy(mfi_t.at[idx_i.at[j]], buf_mfi.at[dst], sem))
        copies.append(pltpu.async_copy(mlpu_t.at[idx_u.at[j]], buf_mlpu.at[dst], sem))
        copies.append(pltpu.async_copy(mlpi_t.at[idx_i.at[j]], buf_mlpi.at[dst], sem))
    for c in copies:
        c.wait()
    row = pl.ds(base, B_PER_W)
    pltpu.sync_copy(buf_mfu, out_mfu.at[row])
    pltpu.sync_copy(buf_mfi, out_mfi.at[row])
    pltpu.sync_copy(buf_mlpu, out_mlpu.at[row])
    pltpu.sync_copy(buf_mlpi, out_mlpi.at[row])


def _sc_gather(user, item, mfu_t, mfi_t, mlpu_t, mlpi_t):
    mesh = plsc.VectorSubcoreMesh(core_axis_name="c", subcore_axis_name="s")
    f32 = jnp.float32
    run = functools.partial(
        pl.kernel,
        mesh=mesh,
        compiler_params=pltpu.CompilerParams(use_tc_tiling_on_sc=False),
        out_type=[
            jax.ShapeDtypeStruct((BATCH, MF_DIM), f32),
            jax.ShapeDtypeStruct((BATCH, MF_DIM), f32),
            jax.ShapeDtypeStruct((BATCH, MLP_HALF), f32),
            jax.ShapeDtypeStruct((BATCH, MLP_HALF), f32),
        ],
        scratch_types=[
            pltpu.VMEM((N_CHUNK, CHUNK), jnp.int32),
            pltpu.VMEM((N_CHUNK, CHUNK), jnp.int32),
            pltpu.VMEM((B_PER_W, MF_DIM), f32),
            pltpu.VMEM((B_PER_W, MF_DIM), f32),
            pltpu.VMEM((B_PER_W, MLP_HALF), f32),
            pltpu.VMEM((B_PER_W, MLP_HALF), f32),
            pltpu.SemaphoreType.DMA,
        ],
    )(_sc_gather_body)
    user_r = user.reshape(NW, N_CHUNK, CHUNK).astype(jnp.int32)
    item_r = item.reshape(NW, N_CHUNK, CHUNK).astype(jnp.int32)
    return run(user_r, item_r, mfu_t, mfi_t, mlpu_t, mlpi_t)


def _tc_mlp_body(xmfu, xmfi, xmlpu, xmlpi, w0u, w0i, b0, w1, b1, w2, b2,
                 w3, b3, wfm, wfp, bf, out):
    f32 = jnp.float32
    xmf = xmfu[...] * xmfi[...]
    h = (jnp.dot(xmlpu[...], w0u[...], preferred_element_type=f32)
         + jnp.dot(xmlpi[...], w0i[...], preferred_element_type=f32)
         + b0[...])
    h = jnp.maximum(h, 0.0)
    for w, b in ((w1, b1), (w2, b2), (w3, b3)):
        h = jnp.maximum(jnp.dot(h, w[...], preferred_element_type=f32) + b[...], 0.0)
    logit = (jnp.dot(xmf, wfm[...], preferred_element_type=f32)
             + jnp.dot(h, wfp[...], preferred_element_type=f32)
             + bf[...])
    out[...] = jax.nn.sigmoid(logit)


def _tc_mlp(xmfu, xmfi, xmlpu, xmlpi, W0, b0, W1, b1, W2, b2, W3, b3, Wf, bf):
    R = 2048
    grid = (BATCH // R,)
    D = MF_DIM
    H = MLP_HALF
    rows = lambda d: pl.BlockSpec((R, d), lambda i: (i, 0))
    full = lambda a, b: pl.BlockSpec((a, b), lambda i: (0, 0))
    in_specs = [
        rows(D), rows(D), rows(H), rows(H),
        full(H, D), full(H, D), full(1, D),
        full(D, D), full(1, D),
        full(D, D), full(1, D),
        full(D, D), full(1, D),
        full(D, 1), full(D, 1), full(1, 1),
    ]
    out_spec = pl.BlockSpec((R, 1), lambda i: (i, 0))
    args = (
        xmfu, xmfi, xmlpu, xmlpi,
        W0[:H], W0[H:], b0.reshape(1, D),
        W1, b1.reshape(1, D),
        W2, b2.reshape(1, D),
        W3, b3.reshape(1, D),
        Wf[:D], Wf[D:], bf.reshape(1, 1),
    )
    return pl.pallas_call(
        _tc_mlp_body,
        grid=grid,
        in_specs=in_specs,
        out_specs=out_spec,
        out_shape=jax.ShapeDtypeStruct((BATCH, 1), jnp.float32),
    )(*args)


def kernel(user, item, mf_user_embed, mf_item_embed, mlp_user_embed,
           mlp_item_embed, W0, b0, W1, b1, W2, b2, W3, b3, Wf, bf):
    xmfu, xmfi, xmlpu, xmlpi = _sc_gather(
        user, item, mf_user_embed, mf_item_embed, mlp_user_embed, mlp_item_embed)
    return _tc_mlp(xmfu, xmfi, xmlpu, xmlpi,
                   W0, b0, W1, b1, W2, b2, W3, b3, Wf, bf)


# speedup vs baseline: 1.4010x; 1.4010x over previous
import functools

import jax
import jax.numpy as jnp
from jax import lax
from jax.experimental import pallas as pl
from jax.experimental.pallas import tpu as pltpu
from jax.experimental.pallas import tpu_sc as plsc

BATCH = 16384
MF_DIM = 32
MLP_HALF = 16
NC = 2
NS = 16
NW = NC * NS
B_PER_W = BATCH // NW
CHUNK = 128
N_CHUNK = B_PER_W // CHUNK


def _sc_gather_body(user, item, mfu_t, mfi_t, mlpu_t, mlpi_t,
                    out_mfu, out_mfi, out_mlpu, out_mlpi,
                    idx_u, idx_i, buf_mfu, buf_mfi, buf_mlpu, buf_mlpi, sem):
    wid = lax.axis_index("s") * NC + lax.axis_index("c")
    base = wid * B_PER_W
    pltpu.sync_copy(user.at[pl.ds(base, B_PER_W)], idx_u)
    pltpu.sync_copy(item.at[pl.ds(base, B_PER_W)], idx_i)
    for c in range(N_CHUNK):
        off = c * CHUNK

        def enq(g, _):
            vu = idx_u[pl.ds(off + g * 16, 16)]
            vi = idx_i[pl.ds(off + g * 16, 16)]
            for k in range(16):
                iu = vu[k]
                ii = vi[k]
                row = pl.ds(g * 16 + k, 1)
                pltpu.async_copy(mfu_t.at[pl.ds(iu, 1)], buf_mfu.at[row], sem)
                pltpu.async_copy(mfi_t.at[pl.ds(ii, 1)], buf_mfi.at[row], sem)
                pltpu.async_copy(mlpu_t.at[pl.ds(iu, 1)], buf_mlpu.at[row], sem)
                pltpu.async_copy(mlpi_t.at[pl.ds(ii, 1)], buf_mlpi.at[row], sem)
            return _

        lax.fori_loop(0, CHUNK // 16, enq, 0)
        pltpu.make_async_copy(mfu_t.at[pl.ds(0, CHUNK)], buf_mfu, sem).wait()
        pltpu.make_async_copy(mfi_t.at[pl.ds(0, CHUNK)], buf_mfi, sem).wait()
        pltpu.make_async_copy(mlpu_t.at[pl.ds(0, CHUNK)], buf_mlpu, sem).wait()
        pltpu.make_async_copy(mlpi_t.at[pl.ds(0, CHUNK)], buf_mlpi, sem).wait()
        row = pl.ds(base + off, CHUNK)
        pltpu.sync_copy(buf_mfu, out_mfu.at[row])
        pltpu.sync_copy(buf_mfi, out_mfi.at[row])
        pltpu.sync_copy(buf_mlpu, out_mlpu.at[row])
        pltpu.sync_copy(buf_mlpi, out_mlpi.at[row])


def _sc_gather(user, item, mfu_t, mfi_t, mlpu_t, mlpi_t):
    mesh = plsc.VectorSubcoreMesh(core_axis_name="c", subcore_axis_name="s")
    f32 = jnp.float32
    run = functools.partial(
        pl.kernel,
        mesh=mesh,
        out_type=[
            jax.ShapeDtypeStruct((BATCH, MF_DIM), f32),
            jax.ShapeDtypeStruct((BATCH, MF_DIM), f32),
            jax.ShapeDtypeStruct((BATCH, MLP_HALF), f32),
            jax.ShapeDtypeStruct((BATCH, MLP_HALF), f32),
        ],
        scratch_types=[
            pltpu.VMEM((B_PER_W,), jnp.int32),
            pltpu.VMEM((B_PER_W,), jnp.int32),
            pltpu.VMEM((CHUNK, MF_DIM), f32),
            pltpu.VMEM((CHUNK, MF_DIM), f32),
            pltpu.VMEM((CHUNK, MLP_HALF), f32),
            pltpu.VMEM((CHUNK, MLP_HALF), f32),
            pltpu.SemaphoreType.DMA,
        ],
    )(_sc_gather_body)
    return run(user, item, mfu_t, mfi_t, mlpu_t, mlpi_t)


def _tc_mlp_body(xmfu, xmfi, xmlpu, xmlpi, w0u, w0i, b0, w1, b1, w2, b2,
                 w3, b3, wfm, wfp, bf, out):
    f32 = jnp.float32
    xmf = xmfu[...] * xmfi[...]
    h = (jnp.dot(xmlpu[...], w0u[...], preferred_element_type=f32)
         + jnp.dot(xmlpi[...], w0i[...], preferred_element_type=f32)
         + b0[...])
    h = jnp.maximum(h, 0.0)
    for w, b in ((w1, b1), (w2, b2), (w3, b3)):
        h = jnp.maximum(jnp.dot(h, w[...], preferred_element_type=f32) + b[...], 0.0)
    logit = (jnp.dot(xmf, wfm[...], preferred_element_type=f32)
             + jnp.dot(h, wfp[...], preferred_element_type=f32)
             + bf[...])
    out[...] = jax.nn.sigmoid(logit)


def _tc_mlp(xmfu, xmfi, xmlpu, xmlpi, W0, b0, W1, b1, W2, b2, W3, b3, Wf, bf):
    R = 2048
    grid = (BATCH // R,)
    D = MF_DIM
    H = MLP_HALF
    rows = lambda d: pl.BlockSpec((R, d), lambda i: (i, 0))
    full = lambda a, b: pl.BlockSpec((a, b), lambda i: (0, 0))
    in_specs = [
        rows(D), rows(D), rows(H), rows(H),
        full(H, D), full(H, D), full(1, D),
        full(D, D), full(1, D),
        full(D, D), full(1, D),
        full(D, D), full(1, D),
        full(D, 1), full(D, 1), full(1, 1),
    ]
    out_spec = pl.BlockSpec((R, 1), lambda i: (i, 0))
    args = (
        xmfu, xmfi, xmlpu, xmlpi,
        W0[:H], W0[H:], b0.reshape(1, D),
        W1, b1.reshape(1, D),
        W2, b2.reshape(1, D),
        W3, b3.reshape(1, D),
        Wf[:D], Wf[D:], bf.reshape(1, 1),
    )
    return pl.pallas_call(
        _tc_mlp_body,
        grid=grid,
        in_specs=in_specs,
        out_specs=out_spec,
        out_shape=jax.ShapeDtypeStruct((BATCH, 1), jnp.float32),
    )(*args)


def kernel(user, item, mf_user_embed, mf_item_embed, mlp_user_embed,
           mlp_item_embed, W0, b0, W1, b1, W2, b2, W3, b3, Wf, bf):
    xmfu, xmfi, xmlpu, xmlpi = _sc_gather(
        user, item, mf_user_embed, mf_item_embed, mlp_user_embed, mlp_item_embed)
    return _tc_mlp(xmfu, xmfi, xmlpu, xmlpi,
                   W0, b0, W1, b1, W2, b2, W3, b3, Wf, bf)
